# baseline (speedup 1.0000x reference)
import functools

import jax
import jax.numpy as jnp
from jax import lax
from jax.experimental import pallas as pl
from jax.experimental.pallas import tpu as pltpu
from jax.experimental.pallas import tpu_sc as plsc

NC = 2
NS = 16
NW = NC * NS
B = 16384
D = 64
L = 16
BPW = B // NW
CH = 128
NCH = BPW // CH

_mesh = plsc.VectorSubcoreMesh(core_axis_name="c", subcore_axis_name="s")

_GATHER_DN = lax.GatherDimensionNumbers(
    offset_dims=(), collapsed_slice_dims=(0,), start_index_map=(0,))


def _perm(v, idx):
    return lax.gather(v, idx[:, None], _GATHER_DN, slice_sizes=(1,),
                      mode=lax.GatherScatterMode.PROMISE_IN_BOUNDS)


@functools.partial(
    pl.kernel,
    mesh=_mesh,
    compiler_params=pltpu.CompilerParams(use_tc_tiling_on_sc=False),
    out_type=jax.ShapeDtypeStruct((B,), jnp.float32),
    scratch_types=[
        pltpu.VMEM((NCH, CH), jnp.int32),
        pltpu.VMEM((NCH, CH), jnp.int32),
        pltpu.VMEM((BPW, D), jnp.float32),
        pltpu.VMEM((BPW, D), jnp.float32),
        pltpu.VMEM((BPW,), jnp.float32),
        pltpu.VMEM((BPW,), jnp.float32),
        pltpu.VMEM((BPW,), jnp.float32),
        pltpu.SemaphoreType.DMA,
    ],
)
def _rec_sc(uid_hbm, fid_hbm, ut_hbm, ft_hbm, ub_hbm, fb_hbm, out_hbm,
            uid_v, fid_v, urows, frows, ubias, fbias, sums, sem):
    wid = lax.axis_index("s") * NC + lax.axis_index("c")
    base = wid * BPW

    pltpu.sync_copy(uid_hbm.at[pl.ds(wid * NCH, NCH)], uid_v)
    pltpu.sync_copy(fid_hbm.at[pl.ds(wid * NCH, NCH)], fid_v)

    copies = []
    for j in range(NCH):
        sl = pl.ds(j * CH, CH)
        copies.append(pltpu.async_copy(ut_hbm.at[uid_v.at[j]], urows.at[sl], sem))
        copies.append(pltpu.async_copy(ft_hbm.at[fid_v.at[j]], frows.at[sl], sem))
        copies.append(pltpu.async_copy(ub_hbm.at[uid_v.at[j]], ubias.at[sl], sem))
        copies.append(pltpu.async_copy(fb_hbm.at[fid_v.at[j]], fbias.at[sl], sem))
    for c in copies:
        c.wait()

    lane_iota = lax.iota(jnp.int32, L)

    def grp_body(g, carry):
        r0 = g * L
        rowsums = jnp.zeros((L,), jnp.float32)
        for k in range(L):
            r = r0 + k
            acc = urows[r, pl.ds(0, L)] * frows[r, pl.ds(0, L)]
            for q in range(1, D // L):
                acc = acc + urows[r, pl.ds(q * L, L)] * frows[r, pl.ds(q * L, L)]
            for sh in (8, 4, 2, 1):
                acc = acc + _perm(acc, lane_iota ^ sh)
            rowsums = jnp.where(lane_iota == k, acc, rowsums)
        sl = pl.ds(r0, L)
        sums[sl] = rowsums + ubias[sl] + fbias[sl]
        return carry

    lax.fori_loop(0, BPW // L, grp_body, 0)

    pltpu.sync_copy(sums, out_hbm.at[pl.ds(base, BPW)])


def kernel(user_id, film_id, user_table, film_table, user_bias_table, film_bias_table):
    uid2d = user_id.astype(jnp.int32).reshape(NW * NCH, CH)
    fid2d = film_id.astype(jnp.int32).reshape(NW * NCH, CH)
    ub = user_bias_table.reshape(-1)
    fb = film_bias_table.reshape(-1)
    return _rec_sc(uid2d, fid2d, user_table, film_table, ub, fb)
